# baseline (speedup 1.0000x reference)
import functools

import jax
import jax.numpy as jnp
from jax import lax
from jax.experimental import pallas as pl
from jax.experimental.pallas import tpu as pltpu
from jax.experimental.pallas import tpu_sc as plsc

V_DIM = 1000000
EMB_DIM = 64
BATCH = 16384
CTX = 20

NC = 2
NS = 16
NW = NC * NS

B_PER_W = BATCH // NW
OUT_PER_CHUNK = 4
ROWS_PER_CHUNK = OUT_PER_CHUNK * CTX
N_CHUNKS = B_PER_W // OUT_PER_CHUNK
NBUF = 4
LANES = 16
COL_GROUPS = EMB_DIM // LANES
INV_CTX = 1.0 / CTX


def _cbow_body(x_hbm, table_hbm, out_hbm, idx_v, bufs, out_v,
               sem0, sem1, sem2, sem3):
    sems = (sem0, sem1, sem2, sem3)
    wid = lax.axis_index("s") * NC + lax.axis_index("c")

    pltpu.sync_copy(x_hbm.at[wid], idx_v)

    def start_gather(c, b):
        pltpu.async_copy(table_hbm.at[idx_v.at[c]], bufs.at[b], sems[b])

    def wait_gather(b):
        pltpu.make_async_copy(
            table_hbm.at[idx_v.at[0]], bufs.at[b], sems[b]).wait()

    def reduce_chunk(c, b):
        buf = bufs.at[b]
        for j in range(OUT_PER_CHUNK):
            out_base = (c * OUT_PER_CHUNK + j) * EMB_DIM
            for g in range(COL_GROUPS):
                acc = buf[j * CTX, pl.ds(g * LANES, LANES)]
                for k in range(1, CTX):
                    acc = acc + buf[j * CTX + k, pl.ds(g * LANES, LANES)]
                out_v[pl.ds(out_base + g * LANES, LANES)] = acc * INV_CTX

    for b in range(NBUF):
        start_gather(b, b)

    @pl.loop(0, N_CHUNKS, step=NBUF)
    def _(cc):
        for b in range(NBUF):
            c = cc + b
            wait_gather(b)
            reduce_chunk(c, b)

            @pl.when(c < N_CHUNKS - NBUF)
            def _():
                start_gather(c + NBUF, b)

    pltpu.sync_copy(out_v, out_hbm.at[pl.ds(wid * B_PER_W * EMB_DIM,
                                            B_PER_W * EMB_DIM)])


@jax.jit
def _cbow_sc(x_grouped, emb_table):
    mesh = plsc.VectorSubcoreMesh(core_axis_name="c", subcore_axis_name="s")
    run = pl.kernel(
        _cbow_body,
        out_type=jax.ShapeDtypeStruct((BATCH * EMB_DIM,), jnp.float32),
        mesh=mesh,
        scratch_types=[
            pltpu.VMEM((N_CHUNKS, ROWS_PER_CHUNK), jnp.int32),
            pltpu.VMEM((NBUF, ROWS_PER_CHUNK, EMB_DIM), jnp.float32),
            pltpu.VMEM((B_PER_W * EMB_DIM,), jnp.float32),
            pltpu.SemaphoreType.DMA,
            pltpu.SemaphoreType.DMA,
            pltpu.SemaphoreType.DMA,
            pltpu.SemaphoreType.DMA,
        ],
        compiler_params=pltpu.CompilerParams(use_tc_tiling_on_sc=False),
    )
    return run(x_grouped, emb_table).reshape(BATCH, EMB_DIM)


def kernel(x, emb_table):
    x_grouped = x.astype(jnp.int32).reshape(NW, N_CHUNKS, ROWS_PER_CHUNK)
    return _cbow_sc(x_grouped, emb_table)
